# speedup vs baseline: 1.1020x; 1.1020x over previous
import jax
import jax.numpy as jnp
from jax.experimental import pallas as pl

_N = 10000
_E = 320000
_D = 128
_BE = 1280


def _node_matmuls_kernel(h_ref, wa_ref, ba_ref, wb_ref, bb_ref, ah_ref, bh_ref):
    h = h_ref[...]
    ah_ref[...] = (
        jnp.dot(h, wa_ref[...], preferred_element_type=jnp.float32) + ba_ref[...]
    )
    bh_ref[...] = (
        jnp.dot(h, wb_ref[...], preferred_element_type=jnp.float32) + bb_ref[...]
    )


def _edge_fwd_kernel(e_ref, ga_ref, gs_ref, wc_ref, bc_ref,
                     msg_ref, r_ref, stats_ref):
    i = pl.program_id(0)
    ce = (
        jnp.dot(e_ref[...], wc_ref[...], preferred_element_type=jnp.float32)
        + bc_ref[...]
    )
    e_ij = ce + gs_ref[...]
    sig = jax.nn.sigmoid(e_ij)
    msg_ref[...] = jnp.maximum(sig * ga_ref[...], 0.0)
    r = jnp.maximum(e_ij, 0.0)
    r_ref[...] = r

    @pl.when(i == 0)
    def _():
        stats_ref[...] = jnp.zeros_like(stats_ref)

    s = jnp.sum(r, axis=0, keepdims=True)
    s2 = jnp.sum(r * r, axis=0, keepdims=True)
    row = jax.lax.broadcasted_iota(jnp.int32, (8, _D), 0)
    upd = jnp.where(row == 0, s, 0.0) + jnp.where(row == 1, s2, 0.0)
    stats_ref[...] += upd


def _node_stage_kernel(h_ref, c_ref, w1_ref, w2_ref, bap_ref, gh_ref, bh_ref,
                       hout_ref):
    h = h_ref[...]
    c = c_ref[...]
    c = jnp.where(jnp.isfinite(c), c, 0.0)
    bundle = (
        jnp.dot(h, w1_ref[...], preferred_element_type=jnp.float32)
        + jnp.dot(c, w2_ref[...], preferred_element_type=jnp.float32)
        + bap_ref[...]
    )
    norm = jnp.maximum(
        jnp.sqrt(jnp.sum(bundle * bundle, axis=1, keepdims=True)), 1e-12
    )
    hn = jnp.maximum(bundle / norm, 0.0)
    mu = jnp.mean(hn, axis=0, keepdims=True)
    var = jnp.mean(jnp.square(hn - mu), axis=0, keepdims=True)
    hn = gh_ref[...] * (hn - mu) / jnp.sqrt(var + 1e-5) + bh_ref[...]
    hout_ref[...] = h + hn


def _edge_out_kernel(e_ref, r_ref, mu_ref, isd_ref, be_ref, out_ref):
    out_ref[...] = (
        e_ref[...]
        + (r_ref[...] - mu_ref[...]) * isd_ref[...]
        + be_ref[...]
    )


def _row(v):
    return v.reshape(1, _D)


@jax.jit
def kernel(h, e, W_A, b_A, W_B, b_B, W_C, b_C, W_apply, b_apply,
           gamma_h, beta_h, gamma_e, beta_e, edge_index):
    src = edge_index[0]
    dst = edge_index[1]

    ah, bh = pl.pallas_call(
        _node_matmuls_kernel,
        out_shape=[
            jax.ShapeDtypeStruct((_N, _D), jnp.float32),
            jax.ShapeDtypeStruct((_N, _D), jnp.float32),
        ],
    )(h, W_A, _row(b_A), W_B, _row(b_B))

    ga = jnp.take(ah, src, axis=0)
    gs = jnp.take(bh, src, axis=0) + jnp.take(bh, dst, axis=0)

    grid_e = _E // _BE
    msg, r, stats = pl.pallas_call(
        _edge_fwd_kernel,
        grid=(grid_e,),
        in_specs=[
            pl.BlockSpec((_BE, _D), lambda i: (i, 0)),
            pl.BlockSpec((_BE, _D), lambda i: (i, 0)),
            pl.BlockSpec((_BE, _D), lambda i: (i, 0)),
            pl.BlockSpec((_D, _D), lambda i: (0, 0)),
            pl.BlockSpec((1, _D), lambda i: (0, 0)),
        ],
        out_specs=[
            pl.BlockSpec((_BE, _D), lambda i: (i, 0)),
            pl.BlockSpec((_BE, _D), lambda i: (i, 0)),
            pl.BlockSpec((8, _D), lambda i: (0, 0)),
        ],
        out_shape=[
            jax.ShapeDtypeStruct((_E, _D), jnp.float32),
            jax.ShapeDtypeStruct((_E, _D), jnp.float32),
            jax.ShapeDtypeStruct((8, _D), jnp.float32),
        ],
    )(e, ga, gs, W_C, _row(b_C))

    c = jax.ops.segment_max(msg, dst, num_segments=_N)

    h_out = pl.pallas_call(
        _node_stage_kernel,
        out_shape=jax.ShapeDtypeStruct((_N, _D), jnp.float32),
    )(h, c, W_apply[:_D], W_apply[_D:], _row(b_apply), _row(gamma_h),
      _row(beta_h))

    s = stats[0:1, :]
    s2 = stats[1:2, :]
    mu = s / _E
    var = s2 / _E - mu * mu
    isd = gamma_e.reshape(1, _D) / jnp.sqrt(var + 1e-5)

    e_out = pl.pallas_call(
        _edge_out_kernel,
        grid=(grid_e,),
        in_specs=[
            pl.BlockSpec((_BE, _D), lambda i: (i, 0)),
            pl.BlockSpec((_BE, _D), lambda i: (i, 0)),
            pl.BlockSpec((1, _D), lambda i: (0, 0)),
            pl.BlockSpec((1, _D), lambda i: (0, 0)),
            pl.BlockSpec((1, _D), lambda i: (0, 0)),
        ],
        out_specs=pl.BlockSpec((_BE, _D), lambda i: (i, 0)),
        out_shape=jax.ShapeDtypeStruct((_E, _D), jnp.float32),
    )(e, r, mu, isd, _row(beta_e))

    return (h_out, e_out)


# speedup vs baseline: 1.9824x; 1.7989x over previous
import jax
import jax.numpy as jnp
from jax import lax
from jax.experimental import pallas as pl
from jax.experimental import pallas as pl_sc
from jax.experimental.pallas import tpu as pltpu
from jax.experimental.pallas import tpu_sc as plsc

_N = 10000
_E = 320000
_D = 128
_BE = 1280

_NW = 32
_PER_W = _E // _NW
_CH = 80
_NCH = _PER_W // _CH


def _sc_gather(ah, bh, src, dst):
    mesh = plsc.VectorSubcoreMesh(core_axis_name="c", subcore_axis_name="s")
    out = jax.ShapeDtypeStruct((_E, _D), jnp.float32)

    @pl.kernel(
        mesh=mesh,
        out_type=[out, out, out],
        scratch_types=[
            pltpu.VMEM((_CH,), jnp.int32),
            pltpu.VMEM((_CH,), jnp.int32),
            pltpu.VMEM((_CH, _D), jnp.float32),
            pltpu.VMEM((_CH, _D), jnp.float32),
            pltpu.VMEM((_CH, _D), jnp.float32),
        ],
    )
    def k(ah_hbm, bh_hbm, src_hbm, dst_hbm, ga_hbm, gbs_hbm, gbd_hbm,
          src_v, dst_v, ga_v, gbs_v, gbd_v):
        wid = lax.axis_index("s") * 2 + lax.axis_index("c")
        base = wid * _PER_W

        @pl.loop(0, _NCH)
        def _(ci):
            off = base + ci * _CH
            pltpu.sync_copy(src_hbm.at[pl.ds(off, _CH)], src_v)
            pltpu.sync_copy(dst_hbm.at[pl.ds(off, _CH)], dst_v)
            pltpu.sync_copy(ah_hbm.at[src_v], ga_v)
            pltpu.sync_copy(bh_hbm.at[src_v], gbs_v)
            pltpu.sync_copy(bh_hbm.at[dst_v], gbd_v)
            pltpu.sync_copy(ga_v, ga_hbm.at[pl.ds(off, _CH)])
            pltpu.sync_copy(gbs_v, gbs_hbm.at[pl.ds(off, _CH)])
            pltpu.sync_copy(gbd_v, gbd_hbm.at[pl.ds(off, _CH)])

    return k(ah, bh, src, dst)


def _node_matmuls_kernel(h_ref, wa_ref, ba_ref, wb_ref, bb_ref, ah_ref, bh_ref):
    h = h_ref[...]
    ah_ref[...] = (
        jnp.dot(h, wa_ref[...], preferred_element_type=jnp.float32) + ba_ref[...]
    )
    bh_ref[...] = (
        jnp.dot(h, wb_ref[...], preferred_element_type=jnp.float32) + bb_ref[...]
    )


def _edge_fwd_kernel(e_ref, ga_ref, gbs_ref, gbd_ref, wc_ref, bc_ref,
                     msg_ref, r_ref, stats_ref):
    i = pl.program_id(0)
    ce = (
        jnp.dot(e_ref[...], wc_ref[...], preferred_element_type=jnp.float32)
        + bc_ref[...]
    )
    e_ij = ce + gbs_ref[...] + gbd_ref[...]
    sig = jax.nn.sigmoid(e_ij)
    msg_ref[...] = jnp.maximum(sig * ga_ref[...], 0.0)
    r = jnp.maximum(e_ij, 0.0)
    r_ref[...] = r

    @pl.when(i == 0)
    def _():
        stats_ref[...] = jnp.zeros_like(stats_ref)

    s = jnp.sum(r, axis=0, keepdims=True)
    s2 = jnp.sum(r * r, axis=0, keepdims=True)
    row = jax.lax.broadcasted_iota(jnp.int32, (8, _D), 0)
    upd = jnp.where(row == 0, s, 0.0) + jnp.where(row == 1, s2, 0.0)
    stats_ref[...] += upd


def _node_stage_kernel(h_ref, c_ref, w1_ref, w2_ref, bap_ref, gh_ref, bh_ref,
                       hout_ref):
    h = h_ref[...]
    c = c_ref[...]
    c = jnp.where(jnp.isfinite(c), c, 0.0)
    bundle = (
        jnp.dot(h, w1_ref[...], preferred_element_type=jnp.float32)
        + jnp.dot(c, w2_ref[...], preferred_element_type=jnp.float32)
        + bap_ref[...]
    )
    norm = jnp.maximum(
        jnp.sqrt(jnp.sum(bundle * bundle, axis=1, keepdims=True)), 1e-12
    )
    hn = jnp.maximum(bundle / norm, 0.0)
    mu = jnp.mean(hn, axis=0, keepdims=True)
    var = jnp.mean(jnp.square(hn - mu), axis=0, keepdims=True)
    hn = gh_ref[...] * (hn - mu) / jnp.sqrt(var + 1e-5) + bh_ref[...]
    hout_ref[...] = h + hn


def _edge_out_kernel(e_ref, r_ref, mu_ref, isd_ref, be_ref, out_ref):
    out_ref[...] = (
        e_ref[...]
        + (r_ref[...] - mu_ref[...]) * isd_ref[...]
        + be_ref[...]
    )


def _row(v):
    return v.reshape(1, _D)


@jax.jit
def kernel(h, e, W_A, b_A, W_B, b_B, W_C, b_C, W_apply, b_apply,
           gamma_h, beta_h, gamma_e, beta_e, edge_index):
    src = edge_index[0]
    dst = edge_index[1]

    ah, bh = pl.pallas_call(
        _node_matmuls_kernel,
        out_shape=[
            jax.ShapeDtypeStruct((_N, _D), jnp.float32),
            jax.ShapeDtypeStruct((_N, _D), jnp.float32),
        ],
    )(h, W_A, _row(b_A), W_B, _row(b_B))

    ga, gbs, gbd = _sc_gather(ah, bh, src, dst)

    grid_e = _E // _BE
    msg, r, stats = pl.pallas_call(
        _edge_fwd_kernel,
        grid=(grid_e,),
        in_specs=[
            pl.BlockSpec((_BE, _D), lambda i: (i, 0)),
            pl.BlockSpec((_BE, _D), lambda i: (i, 0)),
            pl.BlockSpec((_BE, _D), lambda i: (i, 0)),
            pl.BlockSpec((_BE, _D), lambda i: (i, 0)),
            pl.BlockSpec((_D, _D), lambda i: (0, 0)),
            pl.BlockSpec((1, _D), lambda i: (0, 0)),
        ],
        out_specs=[
            pl.BlockSpec((_BE, _D), lambda i: (i, 0)),
            pl.BlockSpec((_BE, _D), lambda i: (i, 0)),
            pl.BlockSpec((8, _D), lambda i: (0, 0)),
        ],
        out_shape=[
            jax.ShapeDtypeStruct((_E, _D), jnp.float32),
            jax.ShapeDtypeStruct((_E, _D), jnp.float32),
            jax.ShapeDtypeStruct((8, _D), jnp.float32),
        ],
    )(e, ga, gbs, gbd, W_C, _row(b_C))

    c = jax.ops.segment_max(msg, dst, num_segments=_N)

    h_out = pl.pallas_call(
        _node_stage_kernel,
        out_shape=jax.ShapeDtypeStruct((_N, _D), jnp.float32),
    )(h, c, W_apply[:_D], W_apply[_D:], _row(b_apply), _row(gamma_h),
      _row(beta_h))

    s = stats[0:1, :]
    s2 = stats[1:2, :]
    mu = s / _E
    var = s2 / _E - mu * mu
    isd = gamma_e.reshape(1, _D) / jnp.sqrt(var + 1e-5)

    e_out = pl.pallas_call(
        _edge_out_kernel,
        grid=(grid_e,),
        in_specs=[
            pl.BlockSpec((_BE, _D), lambda i: (i, 0)),
            pl.BlockSpec((_BE, _D), lambda i: (i, 0)),
            pl.BlockSpec((1, _D), lambda i: (0, 0)),
            pl.BlockSpec((1, _D), lambda i: (0, 0)),
            pl.BlockSpec((1, _D), lambda i: (0, 0)),
        ],
        out_specs=pl.BlockSpec((_BE, _D), lambda i: (i, 0)),
        out_shape=jax.ShapeDtypeStruct((_E, _D), jnp.float32),
    )(e, r, mu, isd, _row(beta_e))

    return (h_out, e_out)
